# baseline (speedup 1.0000x reference)
import functools

import jax
import jax.numpy as jnp
from jax import lax
from jax.experimental import pallas as pl
from jax.experimental.pallas import tpu as pltpu
from jax.experimental.pallas import tpu_sc as plsc

B = 16384
D = 64
NC = 2
NS = 16
NW = NC * NS
BPW = B // NW
CH = 128
NCH = BPW // CH
GRP = 16
NGRP = BPW // GRP
L = 16

_mesh = plsc.VectorSubcoreMesh(core_axis_name="c", subcore_axis_name="s")


@functools.partial(
    pl.kernel,
    mesh=_mesh,
    compiler_params=pltpu.CompilerParams(
        needs_layout_passes=False, use_tc_tiling_on_sc=False),
    out_type=jax.ShapeDtypeStruct((B,), jnp.float32),
    scratch_types=[
        pltpu.VMEM((NCH, CH), jnp.int32),
        pltpu.VMEM((NCH, CH), jnp.int32),
        pltpu.VMEM((BPW, D), jnp.float32),
        pltpu.VMEM((BPW, D), jnp.float32),
        pltpu.VMEM((BPW,), jnp.float32),
        pltpu.SemaphoreType.DMA,
        pltpu.SemaphoreType.DMA,
    ],
)
def _pointmf_sc(user_hbm, item_hbm, eu_hbm, ei_hbm, out_hbm,
                uidx, iidx, urows, irows, outv, sem_u, sem_i):
    wid = lax.axis_index("s") * NC + lax.axis_index("c")
    base = wid * BPW

    pltpu.sync_copy(user_hbm.at[wid], uidx)
    pltpu.sync_copy(item_hbm.at[wid], iidx)

    copies = []
    for j in range(NCH):
        copies.append(pltpu.async_copy(
            eu_hbm.at[uidx.at[j]], urows.at[pl.ds(j * CH, CH)], sem_u))
        copies.append(pltpu.async_copy(
            ei_hbm.at[iidx.at[j]], irows.at[pl.ds(j * CH, CH)], sem_i))
    for c in copies:
        c.wait()

    lanes = lax.iota(jnp.int32, L)

    def group_body(g, carry):
        r0 = g * GRP
        rows = r0 + lanes
        acc = jnp.zeros((L,), jnp.float32)
        for k in range(D):
            col = jnp.full((L,), k, jnp.int32)
            u = plsc.load_gather(urows, [rows, col])
            v = plsc.load_gather(irows, [rows, col])
            acc = acc + u * v
        outv[pl.ds(r0, GRP)] = acc
        return carry

    lax.fori_loop(0, NGRP, group_body, 0)
    pltpu.sync_copy(outv, out_hbm.at[pl.ds(base, BPW)])


def kernel(user, item, embed_user, embed_item):
    u3 = user.reshape(NW, NCH, CH)
    i3 = item.reshape(NW, NCH, CH)
    return _pointmf_sc(u3, i3, embed_user, embed_item)


# speedup vs baseline: 2.2645x; 2.2645x over previous
import functools

import jax
import jax.numpy as jnp
from jax import lax
from jax.experimental import pallas as pl
from jax.experimental.pallas import tpu as pltpu
from jax.experimental.pallas import tpu_sc as plsc

B = 16384
D = 64
SUB = 8
V = 1000000
G = V // SUB
NC = 2
NS = 16
NW = NC * NS
BPW = B // NW
L = 16
NGRP = BPW // L

_mesh = plsc.VectorSubcoreMesh(core_axis_name="c", subcore_axis_name="s")


@functools.partial(
    pl.kernel,
    mesh=_mesh,
    compiler_params=pltpu.CompilerParams(needs_layout_passes=False),
    out_type=jax.ShapeDtypeStruct((B,), jnp.float32),
    scratch_types=[
        pltpu.VMEM((BPW,), jnp.int32),
        pltpu.VMEM((BPW,), jnp.int32),
        pltpu.VMEM((2, L, SUB, D), jnp.float32),
        pltpu.VMEM((2, L, SUB, D), jnp.float32),
        pltpu.VMEM((BPW,), jnp.float32),
        pltpu.SemaphoreType.DMA,
        pltpu.SemaphoreType.DMA,
        pltpu.SemaphoreType.DMA,
        pltpu.SemaphoreType.DMA,
    ],
)
def _pointmf_sc(user_hbm, item_hbm, eu_hbm, ei_hbm, out_hbm,
                uidx, iidx, ubuf, ibuf, outv, su0, su1, si0, si1):
    wid = lax.axis_index("s") * NC + lax.axis_index("c")
    base = wid * BPW

    pltpu.sync_copy(user_hbm.at[pl.ds(base, BPW)], uidx)
    pltpu.sync_copy(item_hbm.at[pl.ds(base, BPW)], iidx)

    lanes = lax.iota(jnp.int32, L)
    sems = (su0, su1, si0, si1)

    def fire(g, slot):
        r0 = g * L
        gu = uidx[pl.ds(r0, L)] >> 3
        gi = iidx[pl.ds(r0, L)] >> 3
        for c in range(L):
            pltpu.async_copy(eu_hbm.at[gu[c]], ubuf.at[slot, c], sems[slot])
            pltpu.async_copy(ei_hbm.at[gi[c]], ibuf.at[slot, c], sems[2 + slot])

    def wait(g, slot):
        for c in range(L):
            pltpu.make_async_copy(
                eu_hbm.at[0], ubuf.at[slot, 0], sems[slot]).wait()
            pltpu.make_async_copy(
                ei_hbm.at[0], ibuf.at[slot, 0], sems[2 + slot]).wait()

    def compute(g, slot):
        r0 = g * L
        su = uidx[pl.ds(r0, L)] & 7
        si = iidx[pl.ds(r0, L)] & 7
        out_vec = jnp.zeros((L,), jnp.float32)
        for c in range(L):
            ju = su[c]
            ji = si[c]
            acc = None
            for k in range(D // L):
                u = ubuf[slot, c, ju, pl.ds(k * L, L)]
                v = ibuf[slot, c, ji, pl.ds(k * L, L)]
                p = u * v
                acc = p if acc is None else acc + p
            csum = plsc.cumsum(acc)
            bs = lax.broadcast(csum[L - 1], (L,))
            out_vec = jnp.where(lanes == c, bs, out_vec)
        outv[pl.ds(r0, L)] = out_vec

    fire(0, 0)

    def pair_body(p, carry):
        g0 = p * 2
        fire(g0 + 1, 1)
        wait(g0, 0)
        compute(g0, 0)

        @pl.when(g0 + 2 < NGRP)
        def _():
            fire(g0 + 2, 0)

        wait(g0 + 1, 1)
        compute(g0 + 1, 1)
        return carry

    lax.fori_loop(0, NGRP // 2, pair_body, 0)
    pltpu.sync_copy(outv, out_hbm.at[pl.ds(base, BPW)])


def kernel(user, item, embed_user, embed_item):
    eu3 = embed_user.reshape(G, SUB, D)
    ei3 = embed_item.reshape(G, SUB, D)
    return _pointmf_sc(user, item, eu3, ei3)
